# baseline (speedup 1.0000x reference)
import functools

import jax
import jax.numpy as jnp
from jax import lax
from jax.experimental import pallas as pl
from jax.experimental.pallas import tpu as pltpu
from jax.experimental.pallas import tpu_sc as plsc

EMBED = 64
BATCH = 16384
TOTAL = 2 * BATCH
NROWS = 1_000_000
CHW = 512
NCHUNK = NROWS // CHW
TAIL_START = NCHUNK * CHW
TAIL_W = NROWS - TAIL_START
LANES = 16

_info = plsc.get_sparse_core_info()
NC, NS = _info.num_cores, _info.num_subcores
NW = NC * NS
NK = 61
NSTAGE = 128
PIECE = 4096

_sc_mesh = plsc.VectorSubcoreMesh(core_axis_name="c", subcore_axis_name="s")


def _splat(x):
    return jnp.broadcast_to(jnp.asarray(x, jnp.int32), (LANES,))


def _c16(v):
    return jnp.full((LANES,), v, jnp.int32)


@functools.partial(
    pl.kernel,
    out_type=jax.ShapeDtypeStruct((TOTAL, EMBED), jnp.float32),
    mesh=_sc_mesh,
    scratch_types=[
        pltpu.VMEM((PIECE,), jnp.int32),
        pltpu.VMEM((TOTAL + LANES,), jnp.int32),
        pltpu.VMEM((EMBED, CHW), jnp.float32),
        pltpu.VMEM((EMBED, CHW), jnp.float32),
        pltpu.VMEM((EMBED, TAIL_W), jnp.float32),
        pltpu.VMEM((NSTAGE, EMBED), jnp.float32),
        pltpu.VMEM((2 * LANES,), jnp.int32),
        pltpu.VMEM((NK + 3,), jnp.int32),
        pltpu.SMEM((NK + 3,), jnp.int32),
        pltpu.SMEM((NK + 3,), jnp.int32),
        pltpu.SemaphoreType.DMA,
        pltpu.SemaphoreType.DMA,
        pltpu.SemaphoreType.DMA,
    ],
    compiler_params=pltpu.CompilerParams(needs_layout_passes=False),
)
def _sc_gather(tableT_hbm, tailT_hbm, idx_hbm, out_hbm,
               piece_v, wl_v, buf_a, buf_b, tail_v, stage_v, temp_v,
               cnt_v, offs_s, cnts_s,
               sem_a, sem_b, sem_out):
    wid = lax.axis_index("s") * NC + lax.axis_index("c")
    iota = lax.iota(jnp.int32, LANES)
    wid_s = _splat(wid)
    lane0 = iota == _c16(0)
    NKP = NK + 3

    for g in range(NKP // LANES):
        cnt_v[pl.ds(g * LANES, LANES)] = _c16(0)

    def count_piece(p, _):
        pltpu.sync_copy(idx_hbm.at[pl.ds(p * PIECE, PIECE)], piece_v)

        def group(j, carry):
            v = piece_v[pl.ds(j * LANES, LANES)]
            cv = lax.shift_right_logical(v, _c16(9))
            mask = (cv & _c16(31)) == wid_s
            k_local = lax.shift_right_logical(cv, _c16(5))
            mi = jnp.where(mask, _c16(1), _c16(0))
            plsc.addupdate_scatter(cnt_v, [k_local], mi, mask=mask)
            return carry

        return lax.fori_loop(0, PIECE // LANES, group, 0)

    for p in range(TOTAL // PIECE):
        count_piece(p, 0)

    running = jnp.int32(0)
    for g in range(NKP // LANES):
        vec = cnt_v[pl.ds(g * LANES, LANES)]
        pref = plsc.cumsum(vec)
        offs_vec = _splat(running) + pref - vec
        for m in range(LANES):
            offs_s[g * LANES + m] = offs_vec[m]
            cnts_s[g * LANES + m] = vec[m]
        running = running + pref[LANES - 1]
    n_w = running

    def place_piece(p, _):
        pltpu.sync_copy(idx_hbm.at[pl.ds(p * PIECE, PIECE)], piece_v)

        def group(j, carry):
            v = piece_v[pl.ds(j * LANES, LANES)]
            cv = lax.shift_right_logical(v, _c16(9))
            mask = (cv & _c16(31)) == wid_s
            k_local = lax.shift_right_logical(cv, _c16(5))
            col = v & _c16(CHW - 1)
            b = _splat(p * PIECE + j * LANES) + iota
            key = (
                lax.shift_left(k_local, _c16(24))
                | lax.shift_left(col, _c16(15))
                | b
            )
            mi = jnp.where(mask, _c16(1), _c16(0))
            pref = plsc.cumsum(mi)
            plsc.store_scatter(temp_v, [pref - mi], key, mask=mask)
            cnt = pref[LANES - 1]

            def place(e, c2):
                k2 = temp_v[pl.ds(e, LANES)][0]
                kl = lax.shift_right_logical(k2, 24)
                pos = offs_s[kl]
                offs_s[kl] = pos + 1
                plsc.store_scatter(wl_v, [_splat(pos)], _splat(k2), mask=lane0)
                return c2

            return lax.fori_loop(0, cnt, place, carry)

        return lax.fori_loop(0, PIECE // LANES, group, 0)

    for p in range(TOTAL // PIECE):
        place_piece(p, 0)

    def fire_chunk(k, buf, sem):
        coff = pl.multiple_of((wid + jnp.int32(NW * k)) * jnp.int32(CHW), CHW)
        d = pltpu.make_async_copy(tableT_hbm.at[:, pl.ds(coff, CHW)], buf, sem)
        d.start()
        return d

    def process(buf, k_local, fired0):
        end = offs_s[k_local]
        cnt = cnts_s[k_local]
        start = end - cnt

        def entry(e, f):
            key = wl_v[pl.ds(start + e, LANES)][0]
            col = lax.shift_right_logical(key, 15) & (CHW - 1)
            b = key & (TOTAL - 1)
            slot = f & (NSTAGE - 1)

            @pl.when((f >= NSTAGE) & (slot == 0))
            def _():
                pltpu.make_async_copy(
                    out_hbm.at[pl.ds(0, NSTAGE)], stage_v, sem_out
                ).wait()

            col_s = _splat(col)
            slot_s = _splat(slot)
            for g in range(EMBED // LANES):
                rows_g = iota + _c16(g * LANES)
                vals = plsc.load_gather(buf, [rows_g, col_s])
                plsc.store_scatter(stage_v, [slot_s, rows_g], vals)
            pltpu.make_async_copy(
                stage_v.at[slot], out_hbm.at[b], sem_out
            ).start()
            return f + 1

        return lax.fori_loop(0, cnt, entry, fired0)

    fired = jnp.int32(0)
    cur = fire_chunk(0, buf_a, sem_a)
    bufs = (buf_a, buf_b)
    sems = (sem_a, sem_b)
    for k in range(NK):
        nxt = None
        if k + 1 < NK:
            nxt = fire_chunk(k + 1, bufs[(k + 1) % 2], sems[(k + 1) % 2])
        cur.wait()
        fired = process(bufs[k % 2], jnp.int32(k), fired)
        cur = nxt

    @pl.when(wid == 0)
    def _():
        pltpu.sync_copy(tableT_hbm.at[:, pl.ds(NK * NW * CHW, CHW)], buf_a)

    @pl.when(wid == 1)
    def _():
        pltpu.sync_copy(tailT_hbm, tail_v)

    fired = lax.cond(
        wid == 0,
        lambda f: process(buf_a, jnp.int32(NK), f),
        lambda f: lax.cond(
            wid == 1,
            lambda f2: process(tail_v, jnp.int32(NK), f2),
            lambda f2: f2,
            f,
        ),
        fired,
    )

    consumed = lax.select(
        fired >= NSTAGE + 1,
        ((fired - 1) // NSTAGE) * NSTAGE,
        jnp.int32(0),
    )

    def drain_one(i, _):
        pltpu.make_async_copy(out_hbm.at[0], stage_v.at[0], sem_out).wait()
        return 0

    lax.fori_loop(0, fired - consumed, drain_one, 0)


def _mlp_body(nm, jb, g4, w1a, w1b, w1c, b1, w2, b2, w3, b3, w4, b4, out):
    f32 = jnp.float32
    h = (
        jnp.dot(nm[...], w1a[...], preferred_element_type=f32)
        + jnp.dot(jb[...], w1b[...], preferred_element_type=f32)
        + jnp.dot(g4[...], w1c[...], preferred_element_type=f32)
        + b1[...]
    )
    h = jnp.maximum(h, 0.0)
    h = jnp.maximum(jnp.dot(h, w2[...], preferred_element_type=f32) + b2[...], 0.0)
    h = jnp.maximum(jnp.dot(h, w3[...], preferred_element_type=f32) + b3[...], 0.0)
    out[...] = jnp.dot(h, w4[...], preferred_element_type=f32) + b4[...]


def kernel(name_idx, job_idx, gender, dob, name_table, W1, b1, W2, b2, W3, b3, W4, b4):
    idx = jnp.concatenate([name_idx, job_idx]).astype(jnp.int32)
    tableT = name_table.T
    tailT = tableT[:, TAIL_START:]
    rows = _sc_gather(tableT, tailT, idx)

    g4 = jnp.concatenate([gender, dob[:, None]], axis=1)
    w1a, w1b, w1c = W1[:EMBED], W1[EMBED : 2 * EMBED], W1[2 * EMBED :]

    BL = 2048
    nblk = BATCH // BL

    def full(a):
        return pl.BlockSpec(a.shape, lambda i: tuple(0 for _ in a.shape))

    b1r, b2r, b3r, b4r = (b.reshape(1, -1) for b in (b1, b2, b3, b4))
    out = pl.pallas_call(
        _mlp_body,
        grid=(nblk,),
        in_specs=[
            pl.BlockSpec((BL, EMBED), lambda i: (i, 0)),
            pl.BlockSpec((BL, EMBED), lambda i: (i + nblk, 0)),
            pl.BlockSpec((BL, 4), lambda i: (i, 0)),
            full(w1a),
            full(w1b),
            full(w1c),
            full(b1r),
            full(W2),
            full(b2r),
            full(W3),
            full(b3r),
            full(W4),
            full(b4r),
        ],
        out_specs=pl.BlockSpec((BL, 1), lambda i: (i, 0)),
        out_shape=jax.ShapeDtypeStruct((BATCH, 1), jnp.float32),
    )(rows, rows, g4, w1a, w1b, w1c, b1r, W2, b2r, W3, b3r, W4, b4r)
    return out


# speedup vs baseline: 1.0791x; 1.0791x over previous
import functools

import jax
import jax.numpy as jnp
from jax import lax
from jax.experimental import pallas as pl
from jax.experimental.pallas import tpu as pltpu
from jax.experimental.pallas import tpu_sc as plsc

EMBED = 64
BATCH = 16384
TOTAL = 2 * BATCH
NROWS = 1_000_000
CHW = 512
NCHUNK = NROWS // CHW
TAIL_START = NCHUNK * CHW
TAIL_W = NROWS - TAIL_START
LANES = 16

_info = plsc.get_sparse_core_info()
NC, NS = _info.num_cores, _info.num_subcores
NW = NC * NS
NK = 61
NSTAGE = 128
PIECE = 4096

_sc_mesh = plsc.VectorSubcoreMesh(core_axis_name="c", subcore_axis_name="s")


def _splat(x):
    return jnp.broadcast_to(jnp.asarray(x, jnp.int32), (LANES,))


def _c16(v):
    return jnp.full((LANES,), v, jnp.int32)


@functools.partial(
    pl.kernel,
    out_type=jax.ShapeDtypeStruct((TOTAL, EMBED), jnp.float32),
    mesh=_sc_mesh,
    scratch_types=[
        pltpu.VMEM((PIECE,), jnp.int32),
        pltpu.VMEM((TOTAL + LANES,), jnp.int32),
        pltpu.VMEM((EMBED, CHW), jnp.float32),
        pltpu.VMEM((EMBED, CHW), jnp.float32),
        pltpu.VMEM((EMBED, TAIL_W), jnp.float32),
        pltpu.VMEM((NSTAGE, EMBED), jnp.float32),
        pltpu.VMEM((2 * LANES,), jnp.int32),
        pltpu.SemaphoreType.DMA,
        pltpu.SemaphoreType.DMA,
        pltpu.SemaphoreType.DMA,
    ],
    compiler_params=pltpu.CompilerParams(needs_layout_passes=False),
)
def _sc_gather(tableT_hbm, tailT_hbm, idx_hbm, out_hbm,
               piece_v, wl_v, buf_a, buf_b, tail_v, stage_v, temp_v,
               sem_a, sem_b, sem_out):
    wid = lax.axis_index("s") * NC + lax.axis_index("c")
    iota = lax.iota(jnp.int32, LANES)
    wid_s = _splat(wid)

    def _fire0(k, buf, sem):
        coff = pl.multiple_of((wid + jnp.int32(NW * k)) * jnp.int32(CHW), CHW)
        d = pltpu.make_async_copy(tableT_hbm.at[:, pl.ds(coff, CHW)], buf, sem)
        d.start()
        return d

    d0 = _fire0(0, buf_a, sem_a)
    d1 = _fire0(1, buf_b, sem_b)

    def build_piece(p, n_w):
        pltpu.sync_copy(idx_hbm.at[pl.ds(p * PIECE, PIECE)], piece_v)

        def group(j, off):
            v = piece_v[pl.ds(j * LANES, LANES)]
            cv = lax.shift_right_logical(v, _c16(9))
            mask = (cv & _c16(31)) == wid_s
            k_local = lax.shift_right_logical(cv, _c16(5))
            col = v & _c16(CHW - 1)
            b = _splat(p * PIECE + j * LANES) + iota
            key = (
                lax.shift_left(k_local, _c16(24))
                | lax.shift_left(col, _c16(15))
                | b
            )
            mi = jnp.where(mask, _c16(1), _c16(0))
            pref = plsc.cumsum(mi)
            pos = _splat(off) + pref - mi
            plsc.store_scatter(wl_v, [pos], key, mask=mask)
            return off + pref[LANES - 1]

        return lax.fori_loop(0, PIECE // LANES, group, n_w)

    n_w = jnp.int32(0)
    for p in range(TOTAL // PIECE):
        n_w = build_piece(p, n_w)

    def fire_chunk(k, buf, sem):
        coff = pl.multiple_of((wid + jnp.int32(NW * k)) * jnp.int32(CHW), CHW)
        d = pltpu.make_async_copy(tableT_hbm.at[:, pl.ds(coff, CHW)], buf, sem)
        d.start()
        return d

    def process(buf, k_local, fired0):
        n_groups = lax.shift_right_logical(n_w + (LANES - 1), 4)
        kk_s = _splat(k_local)

        def group(j, fired):
            wlvec = wl_v[pl.ds(j * LANES, LANES)]
            valid = (j * LANES + iota) < _splat(n_w)
            kv = lax.shift_right_logical(wlvec, _c16(24))
            mask = (kv == kk_s) & valid
            mi = jnp.where(mask, _c16(1), _c16(0))
            pref = plsc.cumsum(mi)
            plsc.store_scatter(temp_v, [pref - mi], wlvec, mask=mask)
            cnt = pref[LANES - 1]

            def entry(e, f):
                key = temp_v[pl.ds(e, LANES)][0]
                col = lax.shift_right_logical(key, 15) & (CHW - 1)
                b = key & (TOTAL - 1)
                slot = f & (NSTAGE - 1)

                @pl.when((f >= NSTAGE) & (slot == 0))
                def _():
                    pltpu.make_async_copy(
                        out_hbm.at[pl.ds(0, NSTAGE)], stage_v, sem_out
                    ).wait()

                col_s = _splat(col)
                slot_s = _splat(slot)
                for g in range(EMBED // LANES):
                    rows_g = iota + _c16(g * LANES)
                    vals = plsc.load_gather(buf, [rows_g, col_s])
                    plsc.store_scatter(stage_v, [slot_s, rows_g], vals)
                pltpu.make_async_copy(
                    stage_v.at[slot], out_hbm.at[b], sem_out
                ).start()
                return f + 1

            return lax.fori_loop(0, cnt, entry, fired)

        return lax.fori_loop(0, n_groups, group, fired0)

    fired = jnp.int32(0)
    bufs = (buf_a, buf_b)
    sems = (sem_a, sem_b)
    cur, nxt = d0, d1
    for k in range(NK):
        cur.wait()
        fired = process(bufs[k % 2], jnp.int32(k), fired)
        cur = nxt
        if k + 2 < NK:
            nxt = fire_chunk(k + 2, bufs[k % 2], sems[k % 2])
        else:
            nxt = None

    @pl.when(wid == 0)
    def _():
        pltpu.sync_copy(tableT_hbm.at[:, pl.ds(NK * NW * CHW, CHW)], buf_a)

    @pl.when(wid == 1)
    def _():
        pltpu.sync_copy(tailT_hbm, tail_v)

    fired = lax.cond(
        wid == 0,
        lambda f: process(buf_a, jnp.int32(NK), f),
        lambda f: lax.cond(
            wid == 1,
            lambda f2: process(tail_v, jnp.int32(NK), f2),
            lambda f2: f2,
            f,
        ),
        fired,
    )

    consumed = lax.select(
        fired >= NSTAGE + 1,
        ((fired - 1) // NSTAGE) * NSTAGE,
        jnp.int32(0),
    )

    def drain_one(i, _):
        pltpu.make_async_copy(out_hbm.at[0], stage_v.at[0], sem_out).wait()
        return 0

    lax.fori_loop(0, fired - consumed, drain_one, 0)


def _mlp_body(nm, jb, g4, w1a, w1b, w1c, b1, w2, b2, w3, b3, w4, b4, out):
    f32 = jnp.float32
    h = (
        jnp.dot(nm[...], w1a[...], preferred_element_type=f32)
        + jnp.dot(jb[...], w1b[...], preferred_element_type=f32)
        + jnp.dot(g4[...], w1c[...], preferred_element_type=f32)
        + b1[...]
    )
    h = jnp.maximum(h, 0.0)
    h = jnp.maximum(jnp.dot(h, w2[...], preferred_element_type=f32) + b2[...], 0.0)
    h = jnp.maximum(jnp.dot(h, w3[...], preferred_element_type=f32) + b3[...], 0.0)
    out[...] = jnp.dot(h, w4[...], preferred_element_type=f32) + b4[...]


def kernel(name_idx, job_idx, gender, dob, name_table, W1, b1, W2, b2, W3, b3, W4, b4):
    idx = jnp.concatenate([name_idx, job_idx]).astype(jnp.int32)
    tableT = name_table.T
    tailT = tableT[:, TAIL_START:]
    rows = _sc_gather(tableT, tailT, idx)

    g4 = jnp.concatenate([gender, dob[:, None]], axis=1)
    w1a, w1b, w1c = W1[:EMBED], W1[EMBED : 2 * EMBED], W1[2 * EMBED :]

    BL = 2048
    nblk = BATCH // BL

    def full(a):
        return pl.BlockSpec(a.shape, lambda i: tuple(0 for _ in a.shape))

    b1r, b2r, b3r, b4r = (b.reshape(1, -1) for b in (b1, b2, b3, b4))
    out = pl.pallas_call(
        _mlp_body,
        grid=(nblk,),
        in_specs=[
            pl.BlockSpec((BL, EMBED), lambda i: (i, 0)),
            pl.BlockSpec((BL, EMBED), lambda i: (i + nblk, 0)),
            pl.BlockSpec((BL, 4), lambda i: (i, 0)),
            full(w1a),
            full(w1b),
            full(w1c),
            full(b1r),
            full(W2),
            full(b2r),
            full(W3),
            full(b3r),
            full(W4),
            full(b4r),
        ],
        out_specs=pl.BlockSpec((BL, 1), lambda i: (i, 0)),
        out_shape=jax.ShapeDtypeStruct((BATCH, 1), jnp.float32),
    )(rows, rows, g4, w1a, w1b, w1c, b1r, W2, b2r, W3, b3r, W4, b4r)
    return out
